# baseline (speedup 1.0000x reference)
import functools

import jax
import jax.numpy as jnp
from jax import lax
from jax.experimental import pallas as pl
from jax.experimental.pallas import tpu as pltpu
from jax.experimental.pallas import tpu_sc as plsc

HIDDEN = 64
LANES = 16
PADW = 128
BLK = 128


def kernel(input_id, embedding, pos_code):
    batch, seq = input_id.shape
    vocab = embedding.shape[0]
    info = plsc.get_sparse_core_info()
    nc, ns = info.num_cores, info.num_subcores
    nw = nc * ns
    assert batch == BLK * nw and seq % 4 == 0

    emb_pad = jnp.pad(embedding, ((0, 0), (0, PADW - HIDDEN)))
    idx_t = input_id.T

    mesh = plsc.VectorSubcoreMesh(core_axis_name="c", subcore_axis_name="s")

    @functools.partial(
        pl.kernel,
        out_type=jax.ShapeDtypeStruct((seq, 8, nw, 8, BLK), jnp.float32),
        mesh=mesh,
        compiler_params=pltpu.CompilerParams(use_tc_tiling_on_sc=False,
                                             needs_layout_passes=False),
        scratch_types=[
            pltpu.VMEM((2, BLK), jnp.int32),
            pltpu.VMEM((2, BLK), jnp.int32),
            pltpu.VMEM((BLK, PADW), jnp.float32),
            pltpu.VMEM((BLK, PADW), jnp.float32),
            pltpu.VMEM((8, 8, BLK), jnp.float32),
            pltpu.VMEM((8, 8, BLK), jnp.float32),
            pltpu.VMEM((HIDDEN,), jnp.float32),
            pltpu.SemaphoreType.DMA,
            pltpu.SemaphoreType.DMA,
            pltpu.SemaphoreType.DMA,
            pltpu.SemaphoreType.DMA,
        ],
    )
    def k(idx_hbm, tab_hbm, pc_hbm, out_hbm,
          idx_v0, idx_v1, rows_v0, rows_v1, dst_v0, dst_v1, pos_v,
          gsem0, gsem1, osem0, osem1):
        wid = lax.axis_index("s") * nc + lax.axis_index("c")
        b0 = wid * BLK

        idx_bufs = (idx_v0, idx_v1)
        row_bufs = (rows_v0, rows_v1)
        dst_bufs = (dst_v0, dst_v1)
        gsems = (gsem0, gsem1)
        osems = (osem0, osem1)

        pltpu.sync_copy(pc_hbm.at[0, seq], pos_v)
        pvecs = [pos_v[pl.ds(j * LANES, LANES)] for j in range(HIDDEN // LANES)]
        iota = lax.iota(jnp.int32, LANES)
        i0s = [2 * j + iota // 8 for j in range(HIDDEN // LANES)]
        i1s = [iota % 8 for _ in range(HIDDEN // LANES)]

        def load_idx(c, p):
            pltpu.sync_copy(
                idx_hbm.at[pl.ds(2 * c, 2), pl.ds(b0, BLK)], idx_bufs[p])

        def fire_gather(b, p):
            pltpu.async_copy(tab_hbm.at[idx_bufs[p].at[b]],
                             row_bufs[b], gsems[b])

        def wait_gather(b, p):
            pltpu.make_async_copy(tab_hbm.at[idx_bufs[p].at[b]],
                                  row_bufs[b], gsems[b]).wait()

        def transpose_add(b):
            rows = row_bufs[b]
            dst = dst_bufs[b]

            @plsc.parallel_loop(0, BLK, unroll=2)
            def _(r):
                rvec = jnp.full((LANES,), r, jnp.int32)
                for j in range(HIDDEN // LANES):
                    v = rows[r, pl.ds(j * LANES, LANES)] + pvecs[j]
                    plsc.store_scatter(dst, [i0s[j], i1s[j], rvec], v)

        def fire_out(s, b):
            pltpu.async_copy(dst_bufs[b], out_hbm.at[s, :, wid], osems[b])

        def wait_out(s, b):
            pltpu.make_async_copy(dst_bufs[b], out_hbm.at[s, :, wid],
                                  osems[b]).wait()

        n_chunks = seq // 2

        load_idx(0, 0)
        for b in range(2):
            fire_gather(b, 0)
        load_idx(1, 1)
        for b in range(2):
            wait_gather(b, 0)
            transpose_add(b)
            fire_out(b, b)
            fire_gather(b, 1)

        def chunk_step(c, p, prefetch):
            s0 = 2 * c
            if prefetch:
                load_idx(c + 1, 1 - p)
            for b in range(2):
                s = s0 + b
                wait_gather(b, p)
                wait_out(s - 2, b)
                transpose_add(b)
                fire_out(s, b)
                if prefetch:
                    fire_gather(b, 1 - p)

        def body(i, carry):
            chunk_step(1 + 2 * i, 1, True)
            chunk_step(2 + 2 * i, 0, True)
            return carry

        lax.fori_loop(0, (n_chunks - 2) // 2, body, 0)

        chunk_step(n_chunks - 1, 1, False)
        for b in range(2):
            wait_out(2 * n_chunks - 2 + b, b)

    tmp = k(idx_t, emb_pad, pos_code)
    return tmp.transpose((2, 4, 0, 1, 3)).reshape(batch, seq, HIDDEN)
